# baseline (speedup 1.0000x reference)
import functools

import jax
import jax.numpy as jnp
from jax import lax
from jax.experimental import pallas as pl
from jax.experimental.pallas import tpu as pltpu

N_PAD = 10240
R = 128
C = 2048
NCH = N_PAD // C
K = 24
NSEG = 4
BIG = 1e9
F32 = jnp.float32
E_TOT = K * N_PAD


def _elu(x):
    return jnp.where(x > 0, x, jnp.exp(jnp.minimum(x, 0.0)) - 1.0)


def _mm_body(x_ref, w_ref, b_ref, o_ref):
    o_ref[...] = jnp.dot(x_ref[...], w_ref[...],
                         preferred_element_type=F32) + b_ref[...]


def _mm(x, w, b):
    return pl.pallas_call(
        _mm_body,
        out_shape=jax.ShapeDtypeStruct((x.shape[0], w.shape[1]), F32),
    )(x, w, b.reshape(1, -1))


def _head_body(h_ref, w1_ref, b1_ref, w2_ref, b2_ref, w3_ref, b3_ref, o_ref):
    o = _elu(jnp.dot(h_ref[...], w1_ref[...], preferred_element_type=F32)
             + b1_ref[...])
    o = _elu(jnp.dot(o, w2_ref[...], preferred_element_type=F32) + b2_ref[...])
    o_ref[...] = jnp.dot(o, w3_ref[...], preferred_element_type=F32) + b3_ref[...]


def _sel_body(rng_ref, hrow_ref, sqr_ref, brow_ref, h_ref, sqc_ref, bcol_ref,
              idx_ref, dbuf_ref):
    i = pl.program_id(0)
    c0 = rng_ref[2 * i]
    c1 = rng_ref[2 * i + 1]
    hr = hrow_ref[...]
    sqr = sqr_ref[...]
    br = brow_ref[...]

    FL = C // 128

    def _fold_min(x):
        return jnp.min(x.reshape(R, FL, 128), axis=1)

    def dist_chunk(c, mf):
        hc = h_ref[pl.ds(c * C, C), :]
        dot = lax.dot_general(hr, hc, (((1,), (1,)), ((), ())),
                              preferred_element_type=F32)
        bc = bcol_ref[c]
        d = (sqr + sqc_ref[c]) - 2.0 * dot
        d = d + jnp.where(br != bc, BIG, 0.0)
        d = d + jnp.where(bc < 0, BIG, 0.0)
        dbuf_ref[c] = d
        return jnp.minimum(mf, _fold_min(d))

    m0f = lax.fori_loop(c0, c1, dist_chunk,
                        jnp.full((R, 128), jnp.inf, F32))
    m0 = jnp.min(m0f, axis=1, keepdims=True)

    iota_out = lax.broadcasted_iota(jnp.int32, (R, 32), 1)
    iota_c = lax.broadcasted_iota(jnp.int32, (R, C), 1)

    def extract(e, carry):
        acc, m, last = carry

        def scan_chunk(c, f):
            amf, cntf, mgtf = f
            d = dbuf_ref[c]
            iot = iota_c + c * C
            eq = jnp.logical_and(d == m, iot > last)
            amf = jnp.minimum(amf, _fold_min(
                jnp.where(eq, iot, jnp.int32(2 ** 30))))
            cntf = cntf + jnp.sum(
                jnp.where(eq, 1, 0).reshape(R, FL, 128), axis=1)
            mgtf = jnp.minimum(mgtf, _fold_min(
                jnp.where(d > m, d, jnp.inf)))
            return amf, cntf, mgtf

        amf, cntf, mgtf = lax.fori_loop(
            c0, c1, scan_chunk,
            (jnp.full((R, 128), jnp.int32(2 ** 30), jnp.int32),
             jnp.zeros((R, 128), jnp.int32),
             jnp.full((R, 128), jnp.inf, F32)))
        am = jnp.min(amf, axis=1, keepdims=True)
        cnt = jnp.sum(cntf, axis=1, keepdims=True)
        mgt = jnp.min(mgtf, axis=1, keepdims=True)
        rem = cnt > 1
        m = jnp.where(rem, m, mgt)
        last = jnp.where(rem, am, jnp.int32(-1))
        return jnp.where(iota_out == e, am, acc), m, last

    acc, _, _ = lax.fori_loop(
        0, K, extract,
        (jnp.zeros((R, 32), jnp.int32), m0,
         jnp.full((R, 1), jnp.int32(-1), jnp.int32)))
    idx_ref[...] = acc


def _select(h, sqr, sqc, brow, bcol, ranges):
    nb = N_PAD // R
    grid_spec = pltpu.PrefetchScalarGridSpec(
        num_scalar_prefetch=1,
        grid=(nb,),
        in_specs=[
            pl.BlockSpec((R, 32), lambda i, s: (i, 0)),
            pl.BlockSpec((R, 1), lambda i, s: (i, 0)),
            pl.BlockSpec((R, 1), lambda i, s: (i, 0)),
            pl.BlockSpec((N_PAD, 32), lambda i, s: (0, 0)),
            pl.BlockSpec((NCH, 1, C), lambda i, s: (0, 0, 0)),
            pl.BlockSpec((NCH, 1, C), lambda i, s: (0, 0, 0)),
        ],
        out_specs=pl.BlockSpec((R, 32), lambda i, s: (i, 0)),
        scratch_shapes=[pltpu.VMEM((NCH, R, C), F32)],
    )
    return pl.pallas_call(
        _sel_body,
        grid_spec=grid_spec,
        out_shape=jax.ShapeDtypeStruct((N_PAD, 32), jnp.int32),
    )(ranges, h, sqr, brow, h, sqc.reshape(NCH, 1, C), bcol.reshape(NCH, 1, C))


def _block_ranges(batch_lc, n):
    nb = N_PAD // R
    ids = jnp.arange(NSEG, dtype=jnp.int32)
    starts = jnp.searchsorted(batch_lc, ids, side='left').astype(jnp.int32)
    ends = jnp.searchsorted(batch_lc, ids, side='right').astype(jnp.int32)
    r0 = jnp.minimum(jnp.arange(nb, dtype=jnp.int32) * R, n - 1)
    r1 = jnp.minimum(r0 + (R - 1), n - 1)
    c0 = starts[batch_lc[r0]] // C
    c1 = -(-ends[batch_lc[r1]] // C)
    c1 = jnp.maximum(c1, c0 + 1)
    small = jnp.any((ends - starts) < K)
    c0 = jnp.where(small, 0, c0)
    c1 = jnp.where(small, NCH, c1)
    return jnp.stack([c0, c1], axis=1).reshape(-1).astype(jnp.int32)


_SC_CACHE = {}


def _sc_gather():
    if "fn" in _SC_CACHE:
        return _SC_CACHE["fn"]
    from jax.experimental.pallas import tpu_sc as plsc
    info = plsc.get_sparse_core_info()
    nw = info.num_cores * info.num_subcores
    b_per_w = E_TOT // nw
    sub = 16
    ch = b_per_w // sub
    mesh = plsc.VectorSubcoreMesh(core_axis_name="c", subcore_axis_name="s")

    @functools.partial(
        pl.kernel, mesh=mesh,
        out_type=jax.ShapeDtypeStruct((E_TOT, 128), F32),
        scratch_types=[
            pltpu.VMEM((ch,), jnp.int32),
            pltpu.VMEM((ch, 128), F32),
            pltpu.SemaphoreType.DMA,
        ],
    )
    def gather(table_hbm, idx_hbm, out_hbm, idx_v, rows_v, sem):
        wid = lax.axis_index("s") * info.num_cores + lax.axis_index("c")
        for s in range(sub):
            base = wid * b_per_w + s * ch
            pltpu.sync_copy(idx_hbm.at[pl.ds(base, ch)], idx_v)
            pltpu.async_copy(table_hbm.at[idx_v], rows_v, sem).wait()
            pltpu.sync_copy(rows_v, out_hbm.at[pl.ds(base, ch)])

    _SC_CACHE["fn"] = gather
    return gather


def _edge_body(hrow_ref, g_ref, w_ref, b_ref, o_ref):
    hr = hrow_ref[...]
    w = w_ref[...]
    b = b_ref[...]
    zm = jnp.full((R, 32), -jnp.inf, F32)
    for k in range(K):
        gk = g_ref[k][:, :32]
        feat = jnp.concatenate([hr, gk - hr], axis=1)
        z = jnp.dot(feat, w, preferred_element_type=F32) + b
        zm = jnp.maximum(zm, z)
    o_ref[...] = zm


def _edge_mlp(h, g3, conv_W, conv_b):
    nb = N_PAD // R
    return pl.pallas_call(
        _edge_body,
        grid=(nb,),
        in_specs=[
            pl.BlockSpec((R, 32), lambda i: (i, 0)),
            pl.BlockSpec((K, R, 128), lambda i: (0, i, 0)),
            pl.BlockSpec((64, 32), lambda i: (0, 0)),
            pl.BlockSpec((1, 32), lambda i: (0, 0)),
        ],
        out_specs=pl.BlockSpec((R, 32), lambda i: (i, 0)),
        out_shape=jax.ShapeDtypeStruct((N_PAD, 32), F32),
    )(h, g3, conv_W, conv_b.reshape(1, 32))


def _edgeconv(h, brow, bcol, ranges, conv_W, conv_b):
    sq = jnp.sum(h * h, axis=1)
    idx = _select(h, sq.reshape(N_PAD, 1), sq.reshape(1, N_PAD), brow, bcol,
                  ranges)
    idx_flat = idx[:, :K].T.reshape(-1)
    h128 = jnp.pad(h, ((0, 0), (0, 96)))
    g = _sc_gather()(h128, idx_flat)
    g3 = g.reshape(K, N_PAD, 128)
    z = _edge_mlp(h, g3, conv_W, conv_b)
    return jax.nn.elu(z)


def kernel(x_lc, batch_lc, enc_W1, enc_b1, enc_W2, enc_b2,
           conv1_W, conv1_b, conv2_W, conv2_b, conv3_W, conv3_b,
           out_W1, out_b1, out_W2, out_b2, out_W3, out_b3):
    n = x_lc.shape[0]
    pad = N_PAD - n
    xp = jnp.pad(x_lc, ((0, pad), (0, 0)))
    bf = jnp.pad(batch_lc.astype(F32), (0, pad), constant_values=-1.0)
    brow = bf.reshape(N_PAD, 1)
    bcol = bf.reshape(1, N_PAD)

    ranges = _block_ranges(batch_lc, n)

    h = jax.nn.elu(_mm(xp, enc_W1, enc_b1))
    h = jax.nn.elu(_mm(h, enc_W2, enc_b2))

    h = _edgeconv(h, brow, bcol, ranges, conv1_W, conv1_b)
    h = _edgeconv(h, brow, bcol, ranges, conv2_W, conv2_b)
    h = _edgeconv(h, brow, bcol, ranges, conv3_W, conv3_b)

    o = pl.pallas_call(
        _head_body,
        out_shape=jax.ShapeDtypeStruct((N_PAD, 8), F32),
    )(h, out_W1, out_b1.reshape(1, 32), out_W2, out_b2.reshape(1, 16),
      out_W3, out_b3.reshape(1, 8))
    return (o[:n], batch_lc)


# speedup vs baseline: 3.3831x; 3.3831x over previous
import functools

import jax
import jax.numpy as jnp
from jax import lax
from jax.experimental import pallas as pl
from jax.experimental.pallas import tpu as pltpu

N_PAD = 10240
R = 128
C = 2048
NCH = N_PAD // C
K = 24
NSEG = 4
BIG = 1e9
F32 = jnp.float32
E_TOT = K * N_PAD


def _elu(x):
    return jnp.where(x > 0, x, jnp.exp(jnp.minimum(x, 0.0)) - 1.0)


def _mm_body(x_ref, w_ref, b_ref, o_ref):
    o_ref[...] = jnp.dot(x_ref[...], w_ref[...],
                         preferred_element_type=F32) + b_ref[...]


def _mm(x, w, b):
    return pl.pallas_call(
        _mm_body,
        out_shape=jax.ShapeDtypeStruct((x.shape[0], w.shape[1]), F32),
    )(x, w, b.reshape(1, -1))


def _head_body(h_ref, w1_ref, b1_ref, w2_ref, b2_ref, w3_ref, b3_ref, o_ref):
    o = _elu(jnp.dot(h_ref[...], w1_ref[...], preferred_element_type=F32)
             + b1_ref[...])
    o = _elu(jnp.dot(o, w2_ref[...], preferred_element_type=F32) + b2_ref[...])
    o_ref[...] = jnp.dot(o, w3_ref[...], preferred_element_type=F32) + b3_ref[...]


def _sel_body(rng_ref, hrow_ref, sqr_ref, brow_ref, h_ref, sqc_ref, bcol_ref,
              idx_ref, dbuf_ref):
    i = pl.program_id(0)
    c0 = rng_ref[2 * i]
    c1 = rng_ref[2 * i + 1]
    hr = hrow_ref[...]
    sqr = sqr_ref[...]
    br = brow_ref[...]

    def dist_chunk(c, m):
        hc = h_ref[pl.ds(c * C, C), :]
        dot = lax.dot_general(hr, hc, (((1,), (1,)), ((), ())),
                              preferred_element_type=F32)
        bc = bcol_ref[c]
        d = (sqr + sqc_ref[c]) - 2.0 * dot
        d = d + jnp.where(br != bc, BIG, 0.0)
        d = d + jnp.where(bc < 0, BIG, 0.0)
        dbuf_ref[c] = d
        return jnp.minimum(m, jnp.min(d, axis=1, keepdims=True))

    m0 = lax.fori_loop(c0, c1, dist_chunk, jnp.full((R, 1), jnp.inf, F32))

    iota_out = lax.broadcasted_iota(jnp.int32, (R, 32), 1)
    iota_c = lax.broadcasted_iota(jnp.int32, (R, C), 1)

    def extract(e, carry):
        acc, m = carry

        def amin_chunk(c, am):
            cand = jnp.where(dbuf_ref[c] == m, iota_c + c * C, 2 ** 30)
            return jnp.minimum(am, jnp.min(cand, axis=1, keepdims=True))

        am = lax.fori_loop(c0, c1, amin_chunk,
                           jnp.full((R, 1), jnp.int32(2 ** 30), jnp.int32))

        def upd_chunk(c, m2):
            dc = jnp.where(iota_c + c * C == am, jnp.inf, dbuf_ref[c])
            dbuf_ref[c] = dc
            return jnp.minimum(m2, jnp.min(dc, axis=1, keepdims=True))

        m = lax.fori_loop(c0, c1, upd_chunk, jnp.full((R, 1), jnp.inf, F32))
        return jnp.where(iota_out == e, am, acc), m

    acc, _ = lax.fori_loop(0, K, extract,
                           (jnp.zeros((R, 32), jnp.int32), m0))
    idx_ref[...] = acc


def _select(h, sqr, sqc, brow, bcol, ranges):
    nb = N_PAD // R
    grid_spec = pltpu.PrefetchScalarGridSpec(
        num_scalar_prefetch=1,
        grid=(nb,),
        in_specs=[
            pl.BlockSpec((R, 32), lambda i, s: (i, 0)),
            pl.BlockSpec((R, 1), lambda i, s: (i, 0)),
            pl.BlockSpec((R, 1), lambda i, s: (i, 0)),
            pl.BlockSpec((N_PAD, 32), lambda i, s: (0, 0)),
            pl.BlockSpec((NCH, 1, C), lambda i, s: (0, 0, 0)),
            pl.BlockSpec((NCH, 1, C), lambda i, s: (0, 0, 0)),
        ],
        out_specs=pl.BlockSpec((R, 32), lambda i, s: (i, 0)),
        scratch_shapes=[pltpu.VMEM((NCH, R, C), F32)],
    )
    return pl.pallas_call(
        _sel_body,
        grid_spec=grid_spec,
        out_shape=jax.ShapeDtypeStruct((N_PAD, 32), jnp.int32),
    )(ranges, h, sqr, brow, h, sqc.reshape(NCH, 1, C), bcol.reshape(NCH, 1, C))


def _block_ranges(batch_lc, n):
    nb = N_PAD // R
    ids = jnp.arange(NSEG, dtype=jnp.int32)
    starts = jnp.searchsorted(batch_lc, ids, side='left').astype(jnp.int32)
    ends = jnp.searchsorted(batch_lc, ids, side='right').astype(jnp.int32)
    r0 = jnp.minimum(jnp.arange(nb, dtype=jnp.int32) * R, n - 1)
    r1 = jnp.minimum(r0 + (R - 1), n - 1)
    c0 = starts[batch_lc[r0]] // C
    c1 = -(-ends[batch_lc[r1]] // C)
    c1 = jnp.maximum(c1, c0 + 1)
    small = jnp.any((ends - starts) < K)
    c0 = jnp.where(small, 0, c0)
    c1 = jnp.where(small, NCH, c1)
    return jnp.stack([c0, c1], axis=1).reshape(-1).astype(jnp.int32)


_SC_CACHE = {}


def _sc_gather():
    if "fn" in _SC_CACHE:
        return _SC_CACHE["fn"]
    from jax.experimental.pallas import tpu_sc as plsc
    info = plsc.get_sparse_core_info()
    nw = info.num_cores * info.num_subcores
    b_per_w = E_TOT // nw
    sub = 16
    ch = b_per_w // sub
    mesh = plsc.VectorSubcoreMesh(core_axis_name="c", subcore_axis_name="s")

    @functools.partial(
        pl.kernel, mesh=mesh,
        out_type=jax.ShapeDtypeStruct((E_TOT, 128), F32),
        scratch_types=[
            pltpu.VMEM((ch,), jnp.int32),
            pltpu.VMEM((ch, 128), F32),
            pltpu.SemaphoreType.DMA,
        ],
    )
    def gather(table_hbm, idx_hbm, out_hbm, idx_v, rows_v, sem):
        wid = lax.axis_index("s") * info.num_cores + lax.axis_index("c")
        for s in range(sub):
            base = wid * b_per_w + s * ch
            pltpu.sync_copy(idx_hbm.at[pl.ds(base, ch)], idx_v)
            pltpu.async_copy(table_hbm.at[idx_v], rows_v, sem).wait()
            pltpu.sync_copy(rows_v, out_hbm.at[pl.ds(base, ch)])

    _SC_CACHE["fn"] = gather
    return gather


def _edge_body(hrow_ref, g_ref, w_ref, b_ref, o_ref):
    hr = hrow_ref[...]
    w = w_ref[...]
    b = b_ref[...]
    zm = jnp.full((R, 32), -jnp.inf, F32)
    for k in range(K):
        gk = g_ref[k][:, :32]
        feat = jnp.concatenate([hr, gk - hr], axis=1)
        z = jnp.dot(feat, w, preferred_element_type=F32) + b
        zm = jnp.maximum(zm, z)
    o_ref[...] = zm


def _edge_mlp(h, g3, conv_W, conv_b):
    nb = N_PAD // R
    return pl.pallas_call(
        _edge_body,
        grid=(nb,),
        in_specs=[
            pl.BlockSpec((R, 32), lambda i: (i, 0)),
            pl.BlockSpec((K, R, 128), lambda i: (0, i, 0)),
            pl.BlockSpec((64, 32), lambda i: (0, 0)),
            pl.BlockSpec((1, 32), lambda i: (0, 0)),
        ],
        out_specs=pl.BlockSpec((R, 32), lambda i: (i, 0)),
        out_shape=jax.ShapeDtypeStruct((N_PAD, 32), F32),
    )(h, g3, conv_W, conv_b.reshape(1, 32))


def _edgeconv(h, brow, bcol, ranges, conv_W, conv_b):
    sq = jnp.sum(h * h, axis=1)
    idx = _select(h, sq.reshape(N_PAD, 1), sq.reshape(1, N_PAD), brow, bcol,
                  ranges)
    idx_flat = idx[:, :K].T.reshape(-1)
    h128 = jnp.pad(h, ((0, 0), (0, 96)))
    g = _sc_gather()(h128, idx_flat)
    g3 = g.reshape(K, N_PAD, 128)
    z = _edge_mlp(h, g3, conv_W, conv_b)
    return jax.nn.elu(z)


def kernel(x_lc, batch_lc, enc_W1, enc_b1, enc_W2, enc_b2,
           conv1_W, conv1_b, conv2_W, conv2_b, conv3_W, conv3_b,
           out_W1, out_b1, out_W2, out_b2, out_W3, out_b3):
    n = x_lc.shape[0]
    pad = N_PAD - n
    xp = jnp.pad(x_lc, ((0, pad), (0, 0)))
    bf = jnp.pad(batch_lc.astype(F32), (0, pad), constant_values=-1.0)
    brow = bf.reshape(N_PAD, 1)
    bcol = bf.reshape(1, N_PAD)

    ranges = _block_ranges(batch_lc, n)

    h = jax.nn.elu(_mm(xp, enc_W1, enc_b1))
    h = jax.nn.elu(_mm(h, enc_W2, enc_b2))

    h = _edgeconv(h, brow, bcol, ranges, conv1_W, conv1_b)
    h = _edgeconv(h, brow, bcol, ranges, conv2_W, conv2_b)
    h = _edgeconv(h, brow, bcol, ranges, conv3_W, conv3_b)

    o = pl.pallas_call(
        _head_body,
        out_shape=jax.ShapeDtypeStruct((N_PAD, 8), F32),
    )(h, out_W1, out_b1.reshape(1, 32), out_W2, out_b2.reshape(1, 16),
      out_W3, out_b3.reshape(1, 8))
    return (o[:n], batch_lc)
